# Initial kernel scaffold; baseline (speedup 1.0000x reference)
import jax
import jax.numpy as jnp
from jax.experimental import pallas as pl


def kernel(x, edge_attr, params, edge_index):
    raise NotImplementedError("write your pallas kernel here")


# speedup vs baseline: 1.3973x; 1.3973x over previous
import functools

import jax
import jax.numpy as jnp
from jax import lax
from jax.experimental import pallas as pl
from jax.experimental.pallas import tpu as pltpu
from jax.experimental.pallas import tpu_sc as plsc

_N, _E, _D, _DE, _H, _L = 10000, 160000, 256, 16, 256, 3
_HALF = _D // 2
_NT = 16
_K = 80
_RPT = 640
_RPT_LAST = _N - 15 * _RPT
_EPT = _E // _NT


def _sc_agg_body(h_hbm, ea_hbm, src_hbm, dst_hbm, out_hbm,
                 src_v, dst_v, off_v, rows_v, ea_v, acc_sh, sem):
    c = lax.axis_index("c")
    s = lax.axis_index("s")
    r0 = pl.multiple_of(s * _RPT, 8)
    h0 = pl.multiple_of(c * _N + s * _RPT, 8)

    @pl.when(s < _NT - 1)
    def _():
        pltpu.sync_copy(h_hbm.at[pl.ds(h0, _RPT)], acc_sh.at[pl.ds(r0, _RPT)])

    @pl.when(s == _NT - 1)
    def _():
        pltpu.sync_copy(h_hbm.at[pl.ds(h0, _RPT_LAST)],
                        acc_sh.at[pl.ds(r0, _RPT_LAST)])

    plsc.subcore_barrier()

    e_base = s * _EPT

    def chunk(i, carry):
        e0 = pl.multiple_of(e_base + i * _K, 8)
        ea0 = pl.multiple_of(c * _E + e_base + i * _K, 8)
        pltpu.sync_copy(src_hbm.at[pl.ds(e0, _K)], src_v)
        pltpu.sync_copy(dst_hbm.at[pl.ds(e0, _K)], dst_v)

        def off_body(j, carry2):
            sl = pl.ds(j * 16, 16)
            off_v[sl] = src_v[sl] + c * _N
            return carry2
        lax.fori_loop(0, _K // 16, off_body, 0)

        pltpu.async_copy(h_hbm.at[off_v], rows_v, sem).wait()
        pltpu.sync_copy(ea_hbm.at[pl.ds(ea0, _K)], ea_v)

        def m_body(r, carry2):
            for j in range(_HALF // 16):
                sl = pl.ds(j * 16, 16)
                rows_v[r, sl] = jnp.maximum(rows_v[r, sl] + ea_v[r, sl], 0.0)
            return carry2
        lax.fori_loop(0, _K, m_body, 0)

        pltpu.sync_copy(rows_v, acc_sh.at[dst_v], add=True)
        return carry

    lax.fori_loop(0, _EPT // _K, chunk, 0)
    plsc.subcore_barrier()

    @pl.when(s < _NT - 1)
    def _():
        pltpu.sync_copy(acc_sh.at[pl.ds(r0, _RPT)],
                        out_hbm.at[pl.ds(h0, _RPT)])

    @pl.when(s == _NT - 1)
    def _():
        pltpu.sync_copy(acc_sh.at[pl.ds(r0, _RPT_LAST)],
                        out_hbm.at[pl.ds(h0, _RPT_LAST)])


_sc_agg = functools.partial(
    pl.kernel,
    mesh=plsc.VectorSubcoreMesh(core_axis_name="c", subcore_axis_name="s"),
    out_type=jax.ShapeDtypeStruct((2 * _N, _HALF), jnp.float32),
    scratch_types=[
        pltpu.VMEM((_K,), jnp.int32),
        pltpu.VMEM((_K,), jnp.int32),
        pltpu.VMEM((_K,), jnp.int32),
        pltpu.VMEM((_K, _HALF), jnp.float32),
        pltpu.VMEM((_K, _HALF), jnp.float32),
        pltpu.VMEM_SHARED((_N, _HALF), jnp.float32),
        pltpu.SemaphoreType.DMA,
    ],
)(_sc_agg_body)


_BE = 2000


def _ea_body(attr_ref, we_ref, be_ref, out_ref):
    out_ref[0, 0] = (
        jnp.dot(attr_ref[...], we_ref[0, 0],
                preferred_element_type=jnp.float32)
        + be_ref[0, 0, 0]
    )


def _ea_call(edge_attr, we_s, be_s):
    return pl.pallas_call(
        _ea_body,
        grid=(_L, 2, _E // _BE),
        in_specs=[
            pl.BlockSpec((_BE, _DE), lambda l, c, i: (i, 0)),
            pl.BlockSpec((1, 1, _DE, _HALF), lambda l, c, i: (l, c, 0, 0)),
            pl.BlockSpec((1, 1, 1, _HALF), lambda l, c, i: (l, c, 0, 0)),
        ],
        out_specs=pl.BlockSpec((1, 1, _BE, _HALF),
                               lambda l, c, i: (l, c, i, 0)),
        out_shape=jax.ShapeDtypeStruct((_L, 2, _E, _HALF), jnp.float32),
    )(edge_attr, we_s, be_s)


_BN = 2000


def _mlp_body(z_ref, w1_ref, b1_ref, w2_ref, b2_ref, g_ref, bt_ref, out_ref):
    z = jnp.concatenate([z_ref[0], z_ref[1]], axis=-1)
    a = jnp.maximum(
        jnp.dot(z, w1_ref[...], preferred_element_type=jnp.float32)
        + b1_ref[...], 0.0)
    b = (jnp.dot(a, w2_ref[...], preferred_element_type=jnp.float32)
         + b2_ref[...])
    r = jnp.maximum(b, 0.0)
    mu = jnp.mean(r, axis=-1, keepdims=True)
    var = jnp.mean((r - mu) * (r - mu), axis=-1, keepdims=True)
    y = (r - mu) * lax.rsqrt(var + 1e-5) * g_ref[...] + bt_ref[...]
    out_ref[0] = y[:, :_HALF]
    out_ref[1] = y[:, _HALF:]


def _mlp_call(z2, w1, b1, w2, b2, g, bt):
    return pl.pallas_call(
        _mlp_body,
        grid=(_N // _BN,),
        in_specs=[
            pl.BlockSpec((2, _BN, _HALF), lambda i: (0, i, 0)),
            pl.BlockSpec((_H, _H), lambda i: (0, 0)),
            pl.BlockSpec((_H,), lambda i: (0,)),
            pl.BlockSpec((_H, _H), lambda i: (0, 0)),
            pl.BlockSpec((_H,), lambda i: (0,)),
            pl.BlockSpec((_H,), lambda i: (0,)),
            pl.BlockSpec((_H,), lambda i: (0,)),
        ],
        out_specs=pl.BlockSpec((2, _BN, _HALF), lambda i: (0, i, 0)),
        out_shape=jax.ShapeDtypeStruct((2, _N, _HALF), jnp.float32),
    )(z2, w1, b1, w2, b2, g, bt)


def kernel(x, edge_attr, params, edge_index):
    src = edge_index[0]
    dst = edge_index[1]
    we_s = jnp.stack([p[0] for p in params])
    be_s = jnp.stack([p[1] for p in params])
    w1_s = [p[2] for p in params]
    b1_s = [p[3] for p in params]
    w2_s = [p[4] for p in params]
    b2_s = [p[5] for p in params]
    g_s = [p[6] for p in params]
    bt_s = [p[7] for p in params]

    we_h = we_s.reshape(_L, _DE, 2, _HALF).transpose(0, 2, 1, 3)
    be_h = be_s.reshape(_L, 2, 1, _HALF)
    ea_all = _ea_call(edge_attr, we_h, be_h)

    h = x.reshape(_N, 2, _HALF).transpose(1, 0, 2).reshape(2 * _N, _HALF)
    for l in range(_L):
        ea_l = ea_all[l].reshape(2 * _E, _HALF)
        z = _sc_agg(h, ea_l, src, dst)
        h2 = _mlp_call(z.reshape(2, _N, _HALF),
                       w1_s[l], b1_s[l], w2_s[l], b2_s[l], g_s[l], bt_s[l])
        h = h2.reshape(2 * _N, _HALF)

    return h.reshape(2, _N, _HALF).transpose(1, 0, 2).reshape(_N, _D)


# speedup vs baseline: 1.6674x; 1.1932x over previous
import functools

import jax
import jax.numpy as jnp
from jax import lax
from jax.experimental import pallas as pl
from jax.experimental.pallas import tpu as pltpu
from jax.experimental.pallas import tpu_sc as plsc

_N, _E, _D, _DE, _H, _L = 10000, 160000, 256, 16, 256, 3
_HALF = _D // 2
_NT = 16
_K = 128
_RPT = 640
_RPT_LAST = _N - 15 * _RPT
_EPT = 10240
_NCH = _EPT // _K
_NCH_LAST = (_E - 15 * _EPT) // _K


def _sc_agg_body(h_hbm, ea_hbm, src_hbm, dst_hbm, out_hbm,
                 src_v, dst_v, off_v, rows_v, acc_sh, sem):
    c = lax.axis_index("c")
    s = lax.axis_index("s")
    r0 = pl.multiple_of(s * _RPT, 8)
    h0 = pl.multiple_of(c * _N + s * _RPT, 8)

    @pl.when(s < _NT - 1)
    def _():
        pltpu.sync_copy(h_hbm.at[pl.ds(h0, _RPT)], acc_sh.at[pl.ds(r0, _RPT)])

    @pl.when(s == _NT - 1)
    def _():
        pltpu.sync_copy(h_hbm.at[pl.ds(h0, _RPT_LAST)],
                        acc_sh.at[pl.ds(r0, _RPT_LAST)])

    plsc.subcore_barrier()

    e_base = s * _EPT

    def chunk(i, carry):
        e0 = pl.multiple_of(e_base + i * _K, 8)
        ea0 = pl.multiple_of(c * _E + e_base + i * _K, 8)
        pltpu.sync_copy(src_hbm.at[pl.ds(e0, _K)], src_v)
        pltpu.sync_copy(dst_hbm.at[pl.ds(e0, _K)], dst_v)
        pltpu.sync_copy(ea_hbm.at[pl.ds(ea0, _K)], rows_v)

        def off_body(j, carry2):
            sl = pl.ds(j * 16, 16)
            off_v[sl] = src_v[sl] + c * _N
            return carry2
        lax.fori_loop(0, _K // 16, off_body, 0)

        pltpu.async_copy(h_hbm.at[off_v], rows_v, sem, add=True).wait()

        def m_body(r, carry2):
            for j in range(_HALF // 16):
                sl = pl.ds(j * 16, 16)
                rows_v[r, sl] = jnp.maximum(rows_v[r, sl], 0.0)
            return carry2
        lax.fori_loop(0, _K, m_body, 0)

        pltpu.sync_copy(rows_v, acc_sh.at[dst_v], add=True)
        return carry

    n_chunks = jnp.where(s < _NT - 1, _NCH, _NCH_LAST)
    lax.fori_loop(0, n_chunks, chunk, 0)
    plsc.subcore_barrier()

    @pl.when(s < _NT - 1)
    def _():
        pltpu.sync_copy(acc_sh.at[pl.ds(r0, _RPT)],
                        out_hbm.at[pl.ds(h0, _RPT)])

    @pl.when(s == _NT - 1)
    def _():
        pltpu.sync_copy(acc_sh.at[pl.ds(r0, _RPT_LAST)],
                        out_hbm.at[pl.ds(h0, _RPT_LAST)])


_sc_agg = functools.partial(
    pl.kernel,
    mesh=plsc.VectorSubcoreMesh(core_axis_name="c", subcore_axis_name="s"),
    out_type=jax.ShapeDtypeStruct((2 * _N, _HALF), jnp.float32),
    scratch_types=[
        pltpu.VMEM((_K,), jnp.int32),
        pltpu.VMEM((_K,), jnp.int32),
        pltpu.VMEM((_K,), jnp.int32),
        pltpu.VMEM((_K, _HALF), jnp.float32),
        pltpu.VMEM_SHARED((_N, _HALF), jnp.float32),
        pltpu.SemaphoreType.DMA,
    ],
)(_sc_agg_body)


_BE = 2000


def _ea_body(attr_ref, we_ref, be_ref, out_ref):
    out_ref[0, 0] = (
        jnp.dot(attr_ref[...], we_ref[0, 0],
                preferred_element_type=jnp.float32)
        + be_ref[0, 0, 0]
    )


def _ea_call(edge_attr, we_s, be_s):
    return pl.pallas_call(
        _ea_body,
        grid=(_L, 2, _E // _BE),
        in_specs=[
            pl.BlockSpec((_BE, _DE), lambda l, c, i: (i, 0)),
            pl.BlockSpec((1, 1, _DE, _HALF), lambda l, c, i: (l, c, 0, 0)),
            pl.BlockSpec((1, 1, 1, _HALF), lambda l, c, i: (l, c, 0, 0)),
        ],
        out_specs=pl.BlockSpec((1, 1, _BE, _HALF),
                               lambda l, c, i: (l, c, i, 0)),
        out_shape=jax.ShapeDtypeStruct((_L, 2, _E, _HALF), jnp.float32),
    )(edge_attr, we_s, be_s)


_BN = 2000


def _mlp_body(z_ref, w1_ref, b1_ref, w2_ref, b2_ref, g_ref, bt_ref, out_ref):
    z = jnp.concatenate([z_ref[0], z_ref[1]], axis=-1)
    a = jnp.maximum(
        jnp.dot(z, w1_ref[...], preferred_element_type=jnp.float32)
        + b1_ref[...], 0.0)
    b = (jnp.dot(a, w2_ref[...], preferred_element_type=jnp.float32)
         + b2_ref[...])
    r = jnp.maximum(b, 0.0)
    mu = jnp.mean(r, axis=-1, keepdims=True)
    var = jnp.mean((r - mu) * (r - mu), axis=-1, keepdims=True)
    y = (r - mu) * lax.rsqrt(var + 1e-5) * g_ref[...] + bt_ref[...]
    out_ref[0] = y[:, :_HALF]
    out_ref[1] = y[:, _HALF:]


def _mlp_call(z2, w1, b1, w2, b2, g, bt):
    return pl.pallas_call(
        _mlp_body,
        grid=(_N // _BN,),
        in_specs=[
            pl.BlockSpec((2, _BN, _HALF), lambda i: (0, i, 0)),
            pl.BlockSpec((_H, _H), lambda i: (0, 0)),
            pl.BlockSpec((_H,), lambda i: (0,)),
            pl.BlockSpec((_H, _H), lambda i: (0, 0)),
            pl.BlockSpec((_H,), lambda i: (0,)),
            pl.BlockSpec((_H,), lambda i: (0,)),
            pl.BlockSpec((_H,), lambda i: (0,)),
        ],
        out_specs=pl.BlockSpec((2, _BN, _HALF), lambda i: (0, i, 0)),
        out_shape=jax.ShapeDtypeStruct((2, _N, _HALF), jnp.float32),
    )(z2, w1, b1, w2, b2, g, bt)


def kernel(x, edge_attr, params, edge_index):
    src = edge_index[0]
    dst = edge_index[1]
    we_s = jnp.stack([p[0] for p in params])
    be_s = jnp.stack([p[1] for p in params])
    w1_s = [p[2] for p in params]
    b1_s = [p[3] for p in params]
    w2_s = [p[4] for p in params]
    b2_s = [p[5] for p in params]
    g_s = [p[6] for p in params]
    bt_s = [p[7] for p in params]

    we_h = we_s.reshape(_L, _DE, 2, _HALF).transpose(0, 2, 1, 3)
    be_h = be_s.reshape(_L, 2, 1, _HALF)
    ea_all = _ea_call(edge_attr, we_h, be_h)

    h = x.reshape(_N, 2, _HALF).transpose(1, 0, 2).reshape(2 * _N, _HALF)
    for l in range(_L):
        ea_l = ea_all[l].reshape(2 * _E, _HALF)
        z = _sc_agg(h, ea_l, src, dst)
        h2 = _mlp_call(z.reshape(2, _N, _HALF),
                       w1_s[l], b1_s[l], w2_s[l], b2_s[l], g_s[l], bt_s[l])
        h = h2.reshape(2 * _N, _HALF)

    return h.reshape(2, _N, _HALF).transpose(1, 0, 2).reshape(_N, _D)


# speedup vs baseline: 3.7481x; 2.2479x over previous
import functools

import jax
import jax.numpy as jnp
from jax import lax
from jax.experimental import pallas as pl
from jax.experimental.pallas import tpu as pltpu
from jax.experimental.pallas import tpu_sc as plsc

_N, _E, _D, _DE, _H, _L = 10000, 160000, 256, 16, 256, 3
_HALF = _D // 2
_NT = 16
_K = 128
_RPT = 640
_RPT_LAST = _N - 15 * _RPT
_EPT = 10240
_NCH = _EPT // _K
_NCH_LAST = (_E - 15 * _EPT) // _K


def _sc_agg_body(h_hbm, ea_hbm, src_hbm, dst_hbm, out_hbm,
                 src0, src1, dst0, dst1, off0, off1, rows0, rows1,
                 acc_sh, ld0, ld1, g0, g1, sc0, sc1):
    c = lax.axis_index("c")
    s = lax.axis_index("s")
    r0 = pl.multiple_of(s * _RPT, 8)
    h0 = pl.multiple_of(c * _N + s * _RPT, 8)

    @pl.when(s < _NT - 1)
    def _():
        pltpu.sync_copy(h_hbm.at[pl.ds(h0, _RPT)], acc_sh.at[pl.ds(r0, _RPT)])

    @pl.when(s == _NT - 1)
    def _():
        pltpu.sync_copy(h_hbm.at[pl.ds(h0, _RPT_LAST)],
                        acc_sh.at[pl.ds(r0, _RPT_LAST)])

    plsc.subcore_barrier()

    e_base = s * _EPT
    cN = c * _N
    cE = c * _E
    npairs = jnp.where(s < _NT - 1, _NCH // 2, _NCH_LAST // 2)

    def ld_descs(i, srcb, dstb, rowsb, ldb):
        e0 = pl.multiple_of(e_base + i * _K, 8)
        ea0 = pl.multiple_of(cE + e_base + i * _K, 8)
        return ((src_hbm.at[pl.ds(e0, _K)], srcb, ldb),
                (dst_hbm.at[pl.ds(e0, _K)], dstb, ldb),
                (ea_hbm.at[pl.ds(ea0, _K)], rowsb, ldb))

    def issue_loads(i, srcb, dstb, rowsb, ldb):
        for a, b, sem in ld_descs(i, srcb, dstb, rowsb, ldb):
            pltpu.async_copy(a, b, sem)

    def wait_loads(i, srcb, dstb, rowsb, ldb):
        for a, b, sem in ld_descs(i, srcb, dstb, rowsb, ldb):
            pltpu.make_async_copy(a, b, sem).wait()

    def comp_off(srcb, offb):
        def body(j, carry):
            sl = pl.ds(j * 16, 16)
            offb[sl] = srcb[sl] + cN
            return carry
        lax.fori_loop(0, _K // 16, body, 0)

    def relu(rowsb):
        def body(r, carry):
            for j in range(_HALF // 16):
                sl = pl.ds(j * 16, 16)
                rowsb[r, sl] = jnp.maximum(rowsb[r, sl], 0.0)
            return carry
        lax.fori_loop(0, _K, body, 0)

    def issue_gather(offb, rowsb, gb):
        pltpu.async_copy(h_hbm.at[offb], rowsb, gb, add=True)

    def wait_gather(offb, rowsb, gb):
        pltpu.make_async_copy(h_hbm.at[offb], rowsb, gb).wait()

    def issue_scatter(rowsb, dstb, scb):
        pltpu.async_copy(rowsb, acc_sh.at[dstb], scb, add=True)

    def wait_scatter(rowsb, dstb, scb):
        pltpu.make_async_copy(rowsb, acc_sh.at[dstb], scb).wait()

    issue_loads(0, src0, dst0, rows0, ld0)
    wait_loads(0, src0, dst0, rows0, ld0)
    comp_off(src0, off0)
    issue_gather(off0, rows0, g0)

    def pair(g, carry):
        i1 = 2 * g + 1

        @pl.when(g >= 1)
        def _():
            wait_scatter(rows1, dst1, sc1)

        issue_loads(i1, src1, dst1, rows1, ld1)
        wait_gather(off0, rows0, g0)
        relu(rows0)
        issue_scatter(rows0, dst0, sc0)
        wait_loads(i1, src1, dst1, rows1, ld1)
        comp_off(src1, off1)
        issue_gather(off1, rows1, g1)

        wait_scatter(rows0, dst0, sc0)

        @pl.when(g < npairs - 1)
        def _():
            issue_loads(2 * g + 2, src0, dst0, rows0, ld0)

        wait_gather(off1, rows1, g1)
        relu(rows1)
        issue_scatter(rows1, dst1, sc1)

        @pl.when(g < npairs - 1)
        def _():
            wait_loads(2 * g + 2, src0, dst0, rows0, ld0)
            comp_off(src0, off0)
            issue_gather(off0, rows0, g0)

        return carry

    lax.fori_loop(0, npairs, pair, 0)
    wait_scatter(rows1, dst1, sc1)
    plsc.subcore_barrier()

    @pl.when(s < _NT - 1)
    def _():
        pltpu.sync_copy(acc_sh.at[pl.ds(r0, _RPT)],
                        out_hbm.at[pl.ds(h0, _RPT)])

    @pl.when(s == _NT - 1)
    def _():
        pltpu.sync_copy(acc_sh.at[pl.ds(r0, _RPT_LAST)],
                        out_hbm.at[pl.ds(h0, _RPT_LAST)])


_sc_agg = functools.partial(
    pl.kernel,
    mesh=plsc.VectorSubcoreMesh(core_axis_name="c", subcore_axis_name="s"),
    out_type=jax.ShapeDtypeStruct((2 * _N, _HALF), jnp.float32),
    scratch_types=[
        pltpu.VMEM((_K,), jnp.int32),
        pltpu.VMEM((_K,), jnp.int32),
        pltpu.VMEM((_K,), jnp.int32),
        pltpu.VMEM((_K,), jnp.int32),
        pltpu.VMEM((_K,), jnp.int32),
        pltpu.VMEM((_K,), jnp.int32),
        pltpu.VMEM((_K, _HALF), jnp.float32),
        pltpu.VMEM((_K, _HALF), jnp.float32),
        pltpu.VMEM_SHARED((_N, _HALF), jnp.float32),
        pltpu.SemaphoreType.DMA,
        pltpu.SemaphoreType.DMA,
        pltpu.SemaphoreType.DMA,
        pltpu.SemaphoreType.DMA,
        pltpu.SemaphoreType.DMA,
        pltpu.SemaphoreType.DMA,
    ],
)(_sc_agg_body)


_BE = 2000


def _ea_body(attr_ref, we_ref, be_ref, out_ref):
    out_ref[0] = (
        jnp.dot(attr_ref[...], we_ref[0],
                preferred_element_type=jnp.float32)
        + be_ref[0]
    )


def _ea_call(edge_attr, we_h, be_h):
    return pl.pallas_call(
        _ea_body,
        grid=(2, _E // _BE),
        in_specs=[
            pl.BlockSpec((_BE, _DE), lambda c, i: (i, 0)),
            pl.BlockSpec((1, _DE, _HALF), lambda c, i: (c, 0, 0)),
            pl.BlockSpec((1, 1, _HALF), lambda c, i: (c, 0, 0)),
        ],
        out_specs=pl.BlockSpec((1, _BE, _HALF), lambda c, i: (c, i, 0)),
        out_shape=jax.ShapeDtypeStruct((2, _E, _HALF), jnp.float32),
    )(edge_attr, we_h, be_h)


_BN = 2000


def _mlp_math(z_ref, w1_ref, b1_ref, w2_ref, b2_ref, g_ref, bt_ref):
    z = jnp.concatenate([z_ref[0], z_ref[1]], axis=-1)
    a = jnp.maximum(
        jnp.dot(z, w1_ref[...], preferred_element_type=jnp.float32)
        + b1_ref[...], 0.0)
    b = (jnp.dot(a, w2_ref[...], preferred_element_type=jnp.float32)
         + b2_ref[...])
    r = jnp.maximum(b, 0.0)
    mu = jnp.mean(r, axis=-1, keepdims=True)
    var = jnp.mean((r - mu) * (r - mu), axis=-1, keepdims=True)
    return (r - mu) * lax.rsqrt(var + 1e-5) * g_ref[...] + bt_ref[...]


def _mlp_body_split(z_ref, w1_ref, b1_ref, w2_ref, b2_ref, g_ref, bt_ref,
                    out_ref):
    y = _mlp_math(z_ref, w1_ref, b1_ref, w2_ref, b2_ref, g_ref, bt_ref)
    out_ref[0] = y[:, :_HALF]
    out_ref[1] = y[:, _HALF:]


def _mlp_body_full(z_ref, w1_ref, b1_ref, w2_ref, b2_ref, g_ref, bt_ref,
                   out_ref):
    out_ref[...] = _mlp_math(z_ref, w1_ref, b1_ref, w2_ref, b2_ref,
                             g_ref, bt_ref)


_MLP_IN_SPECS = [
    pl.BlockSpec((2, _BN, _HALF), lambda i: (0, i, 0)),
    pl.BlockSpec((_H, _H), lambda i: (0, 0)),
    pl.BlockSpec((_H,), lambda i: (0,)),
    pl.BlockSpec((_H, _H), lambda i: (0, 0)),
    pl.BlockSpec((_H,), lambda i: (0,)),
    pl.BlockSpec((_H,), lambda i: (0,)),
    pl.BlockSpec((_H,), lambda i: (0,)),
]


def _mlp_call_split(z2, w1, b1, w2, b2, g, bt):
    return pl.pallas_call(
        _mlp_body_split,
        grid=(_N // _BN,),
        in_specs=_MLP_IN_SPECS,
        out_specs=pl.BlockSpec((2, _BN, _HALF), lambda i: (0, i, 0)),
        out_shape=jax.ShapeDtypeStruct((2, _N, _HALF), jnp.float32),
    )(z2, w1, b1, w2, b2, g, bt)


def _mlp_call_full(z2, w1, b1, w2, b2, g, bt):
    return pl.pallas_call(
        _mlp_body_full,
        grid=(_N // _BN,),
        in_specs=_MLP_IN_SPECS,
        out_specs=pl.BlockSpec((_BN, _D), lambda i: (i, 0)),
        out_shape=jax.ShapeDtypeStruct((_N, _D), jnp.float32),
    )(z2, w1, b1, w2, b2, g, bt)


def kernel(x, edge_attr, params, edge_index):
    src = edge_index[0]
    dst = edge_index[1]
    we_s = jnp.stack([p[0] for p in params])
    be_s = jnp.stack([p[1] for p in params])

    we_h = we_s.reshape(_L, _DE, 2, _HALF).transpose(0, 2, 1, 3)
    be_h = be_s.reshape(_L, 2, 1, _HALF)
    ea = [_ea_call(edge_attr, we_h[l], be_h[l]) for l in range(_L)]

    h = x.reshape(_N, 2, _HALF).transpose(1, 0, 2).reshape(2 * _N, _HALF)
    for l in range(_L):
        _, _, w1, b1, w2, b2, g, bt = params[l]
        z = _sc_agg(h, ea[l].reshape(2 * _E, _HALF), src, dst)
        z2 = z.reshape(2, _N, _HALF)
        if l < _L - 1:
            h = _mlp_call_split(z2, w1, b1, w2, b2, g, bt).reshape(
                2 * _N, _HALF)
        else:
            out = _mlp_call_full(z2, w1, b1, w2, b2, g, bt)
    return out
